# baseline (speedup 1.0000x reference)
import functools
import math

import jax
import jax.numpy as jnp
from jax import lax
from jax.experimental import pallas as pl
from jax.experimental.pallas import tpu as pltpu
from jax.experimental.pallas import tpu_sc as plsc

VOCAB = 1_000_000
EMB = 64
SCALE = math.sqrt(EMB)

NC = 2
NS = 16
NF = EMB // NC


@functools.lru_cache(maxsize=None)
def _make_kernel(seq, ntok):
    t_per_w = ntok // NS
    nvb = t_per_w // 128
    mesh = plsc.VectorSubcoreMesh(core_axis_name="c", subcore_axis_name="s")

    @functools.partial(
        pl.kernel,
        mesh=mesh,
        compiler_params=pltpu.CompilerParams(use_tc_tiling_on_sc=False),
        out_type=jax.ShapeDtypeStruct((seq, 8, ntok // 128, EMB // 8, 128),
                                      jnp.float32),
        scratch_types=[
            pltpu.VMEM_SHARED((VOCAB,), jnp.float32),
            pltpu.VMEM((seq, t_per_w), jnp.int32),
            pltpu.VMEM((2, nvb, 128), jnp.float32),
            pltpu.SemaphoreType.DMA,
            pltpu.SemaphoreType.DMA,
            pltpu.SemaphoreType.DMA,
        ],
    )
    def emb_kernel(tok_hbm, tab_hbm, out_hbm, feat_sh, idx_v, dst_v, sem_st,
                   sem_g, sem_o):
        c = lax.axis_index("c")
        sid = lax.axis_index("s")
        tbase = sid * t_per_w
        e0 = c * NF

        pltpu.sync_copy(tok_hbm.at[:, pl.ds(tbase, t_per_w)], idx_v)

        @pl.when(sid == 0)
        def _():
            pltpu.async_copy(tab_hbm.at[e0], feat_sh, sem_st).wait()

        plsc.subcore_barrier()

        def feat_body(k, carry):
            e = e0 + k
            fb = lax.div(e, 8)
            fi = lax.rem(e, 8)

            out_cp = None
            for s in range(seq):
                db = s % 2
                gcps = [
                    pltpu.async_copy(
                        feat_sh.at[idx_v.at[s, pl.ds(j * 128, 128)]],
                        dst_v.at[db, j],
                        sem_g,
                    )
                    for j in range(nvb)
                ]
                for cp in gcps:
                    cp.wait()

                def scale_j(j, c3, _db=db):
                    for q in range(8):
                        dst_v[_db, j, pl.ds(q * 16, 16)] = (
                            dst_v[_db, j, pl.ds(q * 16, 16)] * SCALE
                        )
                    return c3

                lax.fori_loop(0, nvb, scale_j, 0)

                if out_cp is not None:
                    out_cp.wait()
                out_cp = pltpu.async_copy(
                    dst_v.at[db],
                    out_hbm.at[s, fb, pl.ds(sid * nvb, nvb), fi],
                    sem_o,
                )

            out_cp.wait()

            plsc.subcore_barrier()

            @pl.when((sid == 0) & (k + 1 < NF))
            def _():
                pltpu.async_copy(tab_hbm.at[e + 1], feat_sh, sem_st).wait()

            plsc.subcore_barrier()
            return carry

        lax.fori_loop(0, NF, feat_body, 0)

    return emb_kernel


@jax.jit
def kernel(tokens, table):
    ntok, seq = tokens.shape
    tok_t = tokens.T.astype(jnp.int32)
    tab_t = table.T
    out5 = _make_kernel(seq, ntok)(tok_t, tab_t)
    return out5.transpose(2, 4, 0, 1, 3).reshape(ntok, seq, EMB)


# speedup vs baseline: 3.5382x; 3.5382x over previous
import functools
import math

import jax
import jax.numpy as jnp
from jax import lax
from jax.experimental import pallas as pl
from jax.experimental.pallas import tpu as pltpu
from jax.experimental.pallas import tpu_sc as plsc

VOCAB = 1_000_000
EMB = 64
SCALE = math.sqrt(EMB)

NC = 2
NS = 16
NF = EMB // NC

VB_FULL = VOCAB // 128
V_TAIL = VOCAB - VB_FULL * 128
ROW_PAD = VB_FULL * 128 + 128
VB_PER_TILE = (VB_FULL + NS - 1) // NS
CH = 48
N_CH = (VB_PER_TILE + CH - 1) // CH


def _make_detile():
    mesh = plsc.VectorSubcoreMesh(core_axis_name="c", subcore_axis_name="s")

    @functools.partial(
        pl.kernel,
        mesh=mesh,
        out_type=jax.ShapeDtypeStruct((EMB * ROW_PAD,), jnp.float32),
        scratch_types=[
            pltpu.VMEM((8, CH * 128), jnp.float32),
            pltpu.VMEM((8 * CH * 128,), jnp.float32),
            pltpu.SemaphoreType.DMA,
        ],
    )
    def detile_kernel(tab_hbm, tail_hbm, lin_hbm, v_in, v_out, sem):
        c = lax.axis_index("c")
        sid = lax.axis_index("s")
        lo = sid * VB_PER_TILE
        hi = jnp.minimum(lo + VB_PER_TILE, VB_FULL)

        for fb_loc in range(EMB // 8 // NC):
            fb = c * (EMB // 8 // NC) + fb_loc

            def chunk_body(c2, carry, _fb=fb):
                start = jnp.minimum(lo + c2 * CH, hi - CH)
                pltpu.sync_copy(
                    tab_hbm.at[pl.ds(_fb * 8, 8),
                               pl.ds(start * 128, CH * 128)],
                    v_in,
                )
                for fi in range(8):
                    def scale_q(q, c3, _fi=fi):
                        v_out[pl.ds(_fi * CH * 128 + q * 16, 16)] = (
                            v_in[_fi, pl.ds(q * 16, 16)] * SCALE
                        )
                        return c3

                    lax.fori_loop(0, CH * 128 // 16, scale_q, 0, unroll=8)
                ocps = [
                    pltpu.async_copy(
                        v_out.at[pl.ds(fi * CH * 128, CH * 128)],
                        lin_hbm.at[pl.ds(
                            (_fb * 8 + fi) * ROW_PAD + start * 128, CH * 128)],
                        sem,
                    )
                    for fi in range(8)
                ]
                for cp in ocps:
                    cp.wait()
                return carry

            lax.fori_loop(0, N_CH, chunk_body, 0)

        @pl.when(sid == 0)
        def _():
            def tail_body(el, carry):
                e = c * NF + el
                pltpu.sync_copy(
                    tail_hbm.at[pl.ds(e * V_TAIL, V_TAIL)],
                    v_out.at[pl.ds(0, V_TAIL)],
                )
                pltpu.sync_copy(
                    v_out.at[pl.ds(0, V_TAIL)],
                    lin_hbm.at[pl.ds(e * ROW_PAD + VB_FULL * 128, V_TAIL)],
                )
                return carry

            lax.fori_loop(0, NF, tail_body, 0)

    return detile_kernel


def _make_lookup(seq, ntok):
    t_per_w = ntok // NS
    nvb = t_per_w // 128
    mesh = plsc.VectorSubcoreMesh(core_axis_name="c", subcore_axis_name="s")

    @functools.partial(
        pl.kernel,
        mesh=mesh,
        compiler_params=pltpu.CompilerParams(use_tc_tiling_on_sc=False),
        out_type=jax.ShapeDtypeStruct((seq, 8, ntok // 128, EMB // 8, 128),
                                      jnp.float32),
        scratch_types=[
            pltpu.VMEM_SHARED((VOCAB,), jnp.float32),
            pltpu.VMEM((seq, t_per_w), jnp.int32),
            pltpu.VMEM((2, t_per_w), jnp.float32),
            pltpu.SemaphoreType.DMA,
            pltpu.SemaphoreType.DMA,
            pltpu.SemaphoreType.DMA,
        ],
    )
    def lookup_kernel(tok_hbm, lin_hbm, out_hbm, feat_sh, idx_v, dst_v,
                      sem_st, sem_g, sem_o):
        c = lax.axis_index("c")
        sid = lax.axis_index("s")
        e0 = c * NF

        pltpu.sync_copy(tok_hbm.at[:, sid], idx_v)

        @pl.when(sid == 0)
        def _():
            pltpu.async_copy(
                lin_hbm.at[pl.ds(e0 * ROW_PAD, VOCAB)], feat_sh, sem_st
            ).wait()

        plsc.subcore_barrier()

        def feat_body(k, carry):
            e = e0 + k
            fb = lax.div(e, 8)
            fi = lax.rem(e, 8)

            def out_copies(s, db):
                return [
                    pltpu.async_copy(
                        dst_v.at[db, pl.ds(j * 128, 128)],
                        out_hbm.at[s, fb, sid * nvb + j, fi],
                        sem_o,
                    )
                    for j in range(nvb)
                ]

            gcp = [None, None]
            ocp = [[], []]
            for s in range(seq):
                db = s % 2
                for cp in ocp[db]:
                    cp.wait()
                gcp[db] = pltpu.async_copy(
                    feat_sh.at[idx_v.at[s]], dst_v.at[db], sem_g
                )
                if s >= 1:
                    gcp[1 - db].wait()
                    ocp[1 - db] = out_copies(s - 1, 1 - db)
            last = (seq - 1) % 2
            gcp[last].wait()
            ocp[last] = out_copies(seq - 1, last)
            for cp in ocp[0] + ocp[1]:
                cp.wait()

            plsc.subcore_barrier()

            @pl.when((sid == 0) & (k + 1 < NF))
            def _():
                pltpu.async_copy(
                    lin_hbm.at[pl.ds((e + 1) * ROW_PAD, VOCAB)], feat_sh,
                    sem_st,
                ).wait()

            plsc.subcore_barrier()
            return carry

        lax.fori_loop(0, NF, feat_body, 0)

    return lookup_kernel


@functools.lru_cache(maxsize=None)
def _kernels(seq, ntok):
    return _make_detile(), _make_lookup(seq, ntok)


@jax.jit
def kernel(tokens, table):
    ntok, seq = tokens.shape
    detile, lookup = _kernels(seq, ntok)
    tail_lin = (table[VB_FULL * 128:, :].T * SCALE).reshape(-1)
    tab_lin = detile(table.T, tail_lin)
    tok4 = tokens.T.astype(jnp.int32).reshape(seq, NS, ntok // NS)
    out5 = lookup(tok4, tab_lin)
    return out5.transpose(2, 4, 0, 1, 3).reshape(ntok, seq, EMB)


# speedup vs baseline: 4.2048x; 1.1884x over previous
import functools
import math

import jax
import jax.numpy as jnp
from jax import lax
from jax.experimental import pallas as pl
from jax.experimental.pallas import tpu as pltpu
from jax.experimental.pallas import tpu_sc as plsc

VOCAB = 1_000_000
EMB = 64
SCALE = math.sqrt(EMB)

NC = 2
NS = 16
NF = EMB // NC

VB_FULL = VOCAB // 128
V_TAIL = VOCAB - VB_FULL * 128
ROW_PAD = VB_FULL * 128 + 128
VB_PER_TILE = (VB_FULL + NS - 1) // NS
CH = 24
N_CH = (VB_PER_TILE + CH - 1) // CH


def _make_detile():
    mesh = plsc.VectorSubcoreMesh(core_axis_name="c", subcore_axis_name="s")

    @functools.partial(
        pl.kernel,
        mesh=mesh,
        out_type=jax.ShapeDtypeStruct((EMB * ROW_PAD,), jnp.float32),
        scratch_types=[
            pltpu.VMEM((2, 8, CH * 128), jnp.float32),
            pltpu.VMEM((2, 8 * CH * 128), jnp.float32),
            pltpu.SemaphoreType.DMA,
            pltpu.SemaphoreType.DMA,
        ],
    )
    def detile_kernel(tab_hbm, tail_hbm, lin_hbm, v_in, v_out, sem_i, sem_o):
        c = lax.axis_index("c")
        sid = lax.axis_index("s")
        lo = sid * VB_PER_TILE
        hi = jnp.minimum(lo + VB_PER_TILE, VB_FULL)

        NG = (EMB // 8 // NC) * N_CH

        def chunk_params(g):
            fb_loc, c2 = lax.div(g, N_CH), lax.rem(g, N_CH)
            fb = c * (EMB // 8 // NC) + fb_loc
            start = jnp.minimum(lo + c2 * CH, hi - CH)
            return start, fb

        def in_desc(g, db):
            start, fb = chunk_params(g)
            return pltpu.make_async_copy(
                tab_hbm.at[pl.ds(fb * 8, 8), pl.ds(start * 128, CH * 128)],
                v_in.at[db],
                sem_i,
            )

        def out_desc(g, db, fi):
            start, fb = chunk_params(g)
            return pltpu.make_async_copy(
                v_out.at[db, pl.ds(fi * CH * 128, CH * 128)],
                lin_hbm.at[pl.ds((fb * 8 + fi) * ROW_PAD + start * 128,
                                 CH * 128)],
                sem_o,
            )

        in_desc(0, 0).start()

        def chunk_body(g, carry):
            db = lax.rem(g, 2)

            @pl.when(g + 1 < NG)
            def _():
                in_desc(g + 1, 1 - db).start()

            in_desc(g, db).wait()

            @pl.when(g >= 2)
            def _():
                for fi in range(8):
                    out_desc(g - 2, db, fi).wait()

            for fi in range(8):
                def scale_q(q, c3, _fi=fi):
                    v_out[db, pl.ds(_fi * CH * 128 + q * 16, 16)] = (
                        v_in[db, _fi, pl.ds(q * 16, 16)] * SCALE
                    )
                    return c3

                lax.fori_loop(0, CH * 128 // 16, scale_q, 0, unroll=8)

            for fi in range(8):
                out_desc(g, db, fi).start()
            return carry

        lax.fori_loop(0, NG, chunk_body, 0)
        for g in (NG - 2, NG - 1):
            for fi in range(8):
                out_desc(g, g % 2, fi).wait()

        def tail_body(el, carry):
            e = c * NF + el
            pltpu.sync_copy(
                tail_hbm.at[pl.ds(e * V_TAIL, V_TAIL)],
                v_out.at[0, pl.ds(0, V_TAIL)],
            )
            pltpu.sync_copy(
                v_out.at[0, pl.ds(0, V_TAIL)],
                lin_hbm.at[pl.ds(e * ROW_PAD + VB_FULL * 128, V_TAIL)],
            )
            return carry

        lax.fori_loop(sid * (NF // NS), (sid + 1) * (NF // NS), tail_body, 0)

    return detile_kernel


def _make_lookup(seq, ntok):
    t_per_w = ntok // NS
    nvb = t_per_w // 128
    mesh = plsc.VectorSubcoreMesh(core_axis_name="c", subcore_axis_name="s")

    @functools.partial(
        pl.kernel,
        mesh=mesh,
        compiler_params=pltpu.CompilerParams(use_tc_tiling_on_sc=False),
        out_type=jax.ShapeDtypeStruct((seq, 8, ntok // 128, EMB // 8, 128),
                                      jnp.float32),
        scratch_types=[
            pltpu.VMEM_SHARED((VOCAB,), jnp.float32),
            pltpu.VMEM((seq, t_per_w), jnp.int32),
            pltpu.VMEM((4, t_per_w), jnp.float32),
            pltpu.SemaphoreType.DMA,
            pltpu.SemaphoreType.DMA,
            pltpu.SemaphoreType.DMA,
        ],
    )
    def lookup_kernel(tok_hbm, lin_hbm, out_hbm, feat_sh, idx_v, dst_v,
                      sem_st, sem_g, sem_o):
        c = lax.axis_index("c")
        sid = lax.axis_index("s")
        e0 = c * NF

        pltpu.sync_copy(tok_hbm.at[:, sid], idx_v)

        @pl.when(sid == 0)
        def _():
            pltpu.async_copy(
                lin_hbm.at[pl.ds(e0 * ROW_PAD, VOCAB)], feat_sh, sem_st
            ).wait()

        plsc.subcore_barrier()

        def feat_body(k, carry):
            e = e0 + k
            fb = lax.div(e, 8)
            fi = lax.rem(e, 8)

            def out_copy(s, db):
                return [
                    pltpu.async_copy(
                        dst_v.at[db, pl.ds(j * 128, 128)],
                        out_hbm.at[s, fb, sid * nvb + j, fi],
                        sem_o,
                    )
                    for j in range(nvb)
                ]

            NB = 4
            gcp = [None] * NB
            ocp = [None] * NB
            for s in range(seq):
                db = s % NB
                if ocp[db] is not None:
                    for cp in ocp[db]:
                        cp.wait()
                gcp[db] = pltpu.async_copy(
                    feat_sh.at[idx_v.at[s]], dst_v.at[db], sem_g
                )
                if s >= NB - 1:
                    pdb = (s - (NB - 1)) % NB
                    gcp[pdb].wait()
                    ocp[pdb] = out_copy(s - (NB - 1), pdb)
            for t in range(NB - 1):
                s = seq - (NB - 1) + t
                pdb = s % NB
                gcp[pdb].wait()
                ocp[pdb] = out_copy(s, pdb)
            for cps in ocp:
                if cps is not None:
                    for cp in cps:
                        cp.wait()

            plsc.subcore_barrier()

            @pl.when((sid == 0) & (k + 1 < NF))
            def _():
                pltpu.async_copy(
                    lin_hbm.at[pl.ds((e + 1) * ROW_PAD, VOCAB)], feat_sh,
                    sem_st,
                ).wait()

            plsc.subcore_barrier()
            return carry

        lax.fori_loop(0, NF, feat_body, 0)

    return lookup_kernel


@functools.lru_cache(maxsize=None)
def _kernels(seq, ntok):
    return _make_detile(), _make_lookup(seq, ntok)


@jax.jit
def kernel(tokens, table):
    ntok, seq = tokens.shape
    detile, lookup = _kernels(seq, ntok)
    tail_lin = (table[VB_FULL * 128:, :].T * SCALE).reshape(-1)
    tab_lin = detile(table.T, tail_lin)
    tok4 = tokens.T.astype(jnp.int32).reshape(seq, NS, ntok // NS)
    out5 = lookup(tok4, tab_lin)
    return out5.transpose(2, 4, 0, 1, 3).reshape(ntok, seq, EMB)


# speedup vs baseline: 5.8562x; 1.3928x over previous
import functools
import math

import jax
import jax.numpy as jnp
from jax import lax
from jax.experimental import pallas as pl
from jax.experimental.pallas import tpu as pltpu
from jax.experimental.pallas import tpu_sc as plsc

VOCAB = 1_000_000
EMB = 64
SCALE = math.sqrt(EMB)

NC = 2
NS = 16
NF = EMB // NC

VB_FULL = VOCAB // 128
V_TAIL = VOCAB - VB_FULL * 128
ROW_PAD = VB_FULL * 128 + 128
VB_PER_TILE = (VB_FULL + NS - 1) // NS
CH2 = 49152
N_CH2 = (VB_FULL * 128 + CH2 - 1) // CH2


def _make_detile():
    mesh = plsc.VectorSubcoreMesh(core_axis_name="c", subcore_axis_name="s")

    @functools.partial(
        pl.kernel,
        mesh=mesh,
        out_type=jax.ShapeDtypeStruct((EMB * ROW_PAD,), jnp.float32),
        scratch_types=[
            pltpu.VMEM((2, CH2), jnp.float32),
            pltpu.SemaphoreType.DMA,
            pltpu.SemaphoreType.DMA,
        ],
    )
    def detile_kernel(tab_hbm, tail_hbm, lin_hbm, buf, sem_i, sem_o):
        c = lax.axis_index("c")
        sid = lax.axis_index("s")

        def chunk_params(g):
            row_loc, c2 = lax.div(g, N_CH2), lax.rem(g, N_CH2)
            e = c * NF + sid * 2 + row_loc
            start = jnp.minimum(c2 * CH2, VB_FULL * 128 - CH2)
            return e, start

        def in_desc(g, db):
            e, start = chunk_params(g)
            return pltpu.make_async_copy(
                tab_hbm.at[e, pl.ds(start, CH2)], buf.at[db], sem_i,
            )

        def out_desc(g, db):
            e, start = chunk_params(g)
            return pltpu.make_async_copy(
                buf.at[db], lin_hbm.at[pl.ds(e * ROW_PAD + start, CH2)],
                sem_o,
            )

        NG = 2 * N_CH2
        in_desc(0, 0).start()

        def chunk_body(g, carry):
            db = lax.rem(g, 2)
            in_desc(g, db).wait()

            def scale_q(q, c3):
                buf[db, pl.ds(q * 16, 16)] = buf[db, pl.ds(q * 16, 16)] * SCALE
                return c3

            lax.fori_loop(0, CH2 // 16, scale_q, 0, unroll=8)
            out_desc(g, db).start()

            @pl.when(g >= 1)
            def _():
                out_desc(g - 1, 1 - db).wait()

            @pl.when(g + 1 < NG)
            def _():
                in_desc(g + 1, 1 - db).start()
            return carry

        lax.fori_loop(0, NG, chunk_body, 0)
        out_desc(NG - 1, lax.rem(NG - 1, 2)).wait()

        def tail_body(el, carry):
            e = c * NF + el
            pltpu.sync_copy(
                tail_hbm.at[pl.ds(e * V_TAIL, V_TAIL)],
                buf.at[0, pl.ds(0, V_TAIL)],
            )
            pltpu.sync_copy(
                buf.at[0, pl.ds(0, V_TAIL)],
                lin_hbm.at[pl.ds(e * ROW_PAD + VB_FULL * 128, V_TAIL)],
            )
            return carry

        lax.fori_loop(sid * (NF // NS), (sid + 1) * (NF // NS), tail_body, 0)

    return detile_kernel


def _make_lookup(seq, ntok):
    t_per_w = ntok // NS
    nvb = t_per_w // 128
    mesh = plsc.VectorSubcoreMesh(core_axis_name="c", subcore_axis_name="s")

    @functools.partial(
        pl.kernel,
        mesh=mesh,
        compiler_params=pltpu.CompilerParams(use_tc_tiling_on_sc=False),
        out_type=jax.ShapeDtypeStruct((seq, 8, ntok // 128, EMB // 8, 128),
                                      jnp.float32),
        scratch_types=[
            pltpu.VMEM_SHARED((VOCAB,), jnp.float32),
            pltpu.VMEM((seq, t_per_w), jnp.int32),
            pltpu.VMEM((4, t_per_w), jnp.float32),
            pltpu.SemaphoreType.DMA,
            pltpu.SemaphoreType.DMA,
            pltpu.SemaphoreType.DMA,
        ],
    )
    def lookup_kernel(tok_hbm, lin_hbm, out_hbm, feat_sh, idx_v, dst_v,
                      sem_st, sem_g, sem_o):
        c = lax.axis_index("c")
        sid = lax.axis_index("s")
        e0 = c * NF

        pltpu.sync_copy(tok_hbm.at[:, sid], idx_v)

        @pl.when(sid == 0)
        def _():
            pltpu.async_copy(
                lin_hbm.at[pl.ds(e0 * ROW_PAD, VOCAB)], feat_sh, sem_st
            ).wait()

        plsc.subcore_barrier()

        def feat_body(k, carry):
            e = e0 + k
            fb = lax.div(e, 8)
            fi = lax.rem(e, 8)

            def out_copy(s, db):
                return [
                    pltpu.async_copy(
                        dst_v.at[db, pl.ds(j * 128, 128)],
                        out_hbm.at[s, fb, sid * nvb + j, fi],
                        sem_o,
                    )
                    for j in range(nvb)
                ]

            NB = 4
            gcp = [None] * NB
            ocp = [None] * NB
            for s in range(seq):
                db = s % NB
                if ocp[db] is not None:
                    for cp in ocp[db]:
                        cp.wait()
                gcp[db] = pltpu.async_copy(
                    feat_sh.at[idx_v.at[s]], dst_v.at[db], sem_g
                )
                if s >= NB - 1:
                    pdb = (s - (NB - 1)) % NB
                    gcp[pdb].wait()
                    ocp[pdb] = out_copy(s - (NB - 1), pdb)
            for t in range(NB - 1):
                s = seq - (NB - 1) + t
                pdb = s % NB
                gcp[pdb].wait()
                ocp[pdb] = out_copy(s, pdb)
            for cps in ocp:
                if cps is not None:
                    for cp in cps:
                        cp.wait()

            plsc.subcore_barrier()

            @pl.when((sid == 0) & (k + 1 < NF))
            def _():
                pltpu.async_copy(
                    lin_hbm.at[pl.ds((e + 1) * ROW_PAD, VOCAB)], feat_sh,
                    sem_st,
                ).wait()

            plsc.subcore_barrier()
            return carry

        lax.fori_loop(0, NF, feat_body, 0)

    return lookup_kernel


@functools.lru_cache(maxsize=None)
def _kernels(seq, ntok):
    return _make_detile(), _make_lookup(seq, ntok)


@jax.jit
def kernel(tokens, table):
    ntok, seq = tokens.shape
    detile, lookup = _kernels(seq, ntok)
    tail_lin = (table[VB_FULL * 128:, :].T * SCALE).reshape(-1)
    tab_lin = detile(table.T, tail_lin)
    tok4 = tokens.T.astype(jnp.int32).reshape(seq, NS, ntok // NS)
    out5 = lookup(tok4, tab_lin)
    return out5.transpose(2, 4, 0, 1, 3).reshape(ntok, seq, EMB)
